# Initial kernel scaffold; baseline (speedup 1.0000x reference)
import jax
import jax.numpy as jnp
from jax.experimental import pallas as pl


def kernel(x, edge_index, edge_attr, intarna_energy, batch, covalent_edges, dropout_conv_1_2, dropout_conv_rest, params):
    raise NotImplementedError("write your pallas kernel here")


# speedup vs baseline: 2.2876x; 2.2876x over previous
import functools

import numpy as np
import jax
import jax.numpy as jnp
from jax import lax
from jax.experimental import pallas as pl
from jax.experimental.pallas import tpu as pltpu
from jax.experimental.pallas import tpu_sc as plsc

N = 10000
E = 320000
D = 128
G = 64
AVG_LOG = float(np.log(33.0))

NC = 2
NS = 16
NW = NC * NS

NB = 63
R = 160
NPAD = NB * R
SLOTS = 2 * E
SPW = SLOTS // NW
CHUNK = 2000
FL = 128
NFLUSH = 16
CAP = NFLUSH * FL

_i32 = jnp.int32
_f32 = jnp.float32


def _ld_scalar(ref, idx):
    return ref[pl.ds(idx, 16)][0]


def _st_scalar(ref_slice_fn, pos, val):
    base = (pos // 16) * 16
    lane = pos - base
    sl = pl.ds(base, 16)
    old = ref_slice_fn(sl)[...]
    ref_slice_fn(sl)[...] = jnp.where(lax.iota(_i32, 16) == lane, val, old)


def _route_body(ei0, ei1, cov, route, counts,
                e0b, e1b, cvb, w1b, w2b, bkb, actb, mini, cstage, cnts):
    wid = lax.axis_index("s") * NC + lax.axis_index("c")
    is_rev = wid >= 16
    ebase = jnp.where(is_rev, (wid - 16) * SPW, wid * SPW)

    for b in range(NB):
        cnts[b] = 0

    def chunk_body(ci, _):
        off = ebase + ci * CHUNK
        pltpu.sync_copy(ei0.at[pl.ds(off, CHUNK)], e0b)
        pltpu.sync_copy(ei1.at[pl.ds(off, CHUNK)], e1b)
        pltpu.sync_copy(cov.at[pl.ds(off, CHUNK)], cvb)

        rvv = jnp.full((16,), is_rev.astype(_i32))

        def grp_body(j, _):
            sl = pl.ds(j * 16, 16)
            a0 = e0b[sl]
            a1 = e1b[sl]
            cv = cvb[sl]
            src = a0 + (a1 - a0) * rvv
            dst = a1 + (a0 - a1) * rvv
            le = 1 - lax.shift_right_logical(a1 - a0, 31)
            act = (cv | le) + ((1 - cv) * le - (cv | le)) * rvv
            bk = lax.shift_right_logical(dst * 52429, 23)
            dl = dst - bk * R
            w1b[sl] = src | lax.shift_left(dl, 16)
            w2b[sl] = off + j * 16 + lax.iota(_i32, 16)
            bkb[sl] = bk
            actb[sl] = act
            return 0

        lax.fori_loop(0, CHUNK // 16, grp_body, 0)

        def slot_body(i, _):
            @pl.when(_ld_scalar(actb, i) != 0)
            def _():
                b = _ld_scalar(bkb, i)
                c = cnts[b]
                f = lax.div(c, FL)
                pos = lax.rem(c, FL)
                cur = lax.rem(f, 2)
                _st_scalar(lambda sl: mini.at[b, cur, 0, sl], pos,
                           _ld_scalar(w1b, i))
                _st_scalar(lambda sl: mini.at[b, cur, 1, sl], pos,
                           _ld_scalar(w2b, i))
                cnts[b] = c + 1

                @pl.when(pos == FL - 1)
                def _():
                    fc = jnp.minimum(f, NFLUSH - 1)
                    pltpu.sync_copy(mini.at[b, cur], route.at[wid, b, fc])
            return 0

        lax.fori_loop(0, CHUNK, slot_body, 0)
        return 0

    lax.fori_loop(0, SPW // CHUNK, chunk_body, 0)

    for b in range(NB):
        c = cnts[b]
        f = lax.div(c, FL)

        @pl.when(lax.rem(c, FL) > 0)
        def _():
            fc = jnp.minimum(f, NFLUSH - 1)
            pltpu.sync_copy(mini.at[b, lax.rem(f, 2)], route.at[wid, b, fc])
        _st_scalar(lambda sl: cstage.at[sl], b, c)
    _st_scalar(lambda sl: cstage.at[sl], NB, 0)
    pltpu.sync_copy(cstage, counts.at[wid])


def _route_edges(ei0, ei1, cov32):
    mesh = plsc.VectorSubcoreMesh(core_axis_name="c", subcore_axis_name="s", num_cores=NC, num_subcores=NS)
    f = pl.kernel(
        _route_body,
        out_type=[jax.ShapeDtypeStruct((NW, NB, NFLUSH, 2, FL), _i32),
                  jax.ShapeDtypeStruct((NW, 64), _i32)],
        mesh=mesh,
        scratch_types=[
            pltpu.VMEM((CHUNK,), _i32),
            pltpu.VMEM((CHUNK,), _i32),
            pltpu.VMEM((CHUNK,), _i32),
            pltpu.VMEM((CHUNK + 16,), _i32),
            pltpu.VMEM((CHUNK + 16,), _i32),
            pltpu.VMEM((CHUNK + 16,), _i32),
            pltpu.VMEM((CHUNK + 16,), _i32),
            pltpu.VMEM((NB, 2, 2, FL), _i32),
            pltpu.VMEM((64,), _i32),
            pltpu.SMEM((NB,), _i32),
        ],
    )
    return f(ei0, ei1, cov32)


def _acc_body(xr, ep, route, counts, deg, s1, s2, mn, mx,
              cbuf, recbuf, idxb, eidb, dlb, xrb, epb, s1s, s2s, mns, mxs,
              degs, sem1, sem2):
    wid = lax.axis_index("s") * NC + lax.axis_index("c")
    pltpu.sync_copy(counts, cbuf.at[pl.ds(0, NW * 64)])

    for p in range(2):
        b = p * 32 + wid

        @pl.when(b < NB)
        def _():
            def init_body(i, _):
                for k in range(8):
                    sl = pl.ds(k * 16, 16)
                    s1s[i, sl] = jnp.zeros((16,), _f32)
                    s2s[i, sl] = jnp.zeros((16,), _f32)
                    mns[i, sl] = jnp.full((16,), jnp.inf, _f32)
                    mxs[i, sl] = jnp.full((16,), -jnp.inf, _f32)
                return 0

            lax.fori_loop(0, R, init_body, 0)

            def dinit_body(i, _):
                degs[pl.ds(i * 16, 16)] = jnp.zeros((16,), _f32)
                return 0

            lax.fori_loop(0, R // 16, dinit_body, 0)

            def src_body(t, _):
                cnt = jnp.minimum(_ld_scalar(cbuf, t * 64 + b), CAP)
                nfl = lax.div(cnt + FL - 1, FL)

                def flush_body(f, _):
                    m = jnp.minimum(cnt - f * FL, FL)
                    pltpu.sync_copy(route.at[t, b, f], recbuf)

                    def unpack_body(j, _):
                        sl = pl.ds(j * 16, 16)
                        valid = (j * 16 + lax.iota(_i32, 16)) < m
                        w1 = recbuf[0, sl]
                        eidv = recbuf[1, sl]
                        idxb[sl] = jnp.where(valid, w1 & 0xFFFF, 0)
                        eidb[sl] = jnp.where(valid, eidv, 0)
                        dlb[sl] = lax.shift_right_logical(w1, 16)
                        return 0

                    lax.fori_loop(0, FL // 16, unpack_body, 0)
                    c1 = pltpu.async_copy(xr.at[idxb], xrb, sem1)
                    c2 = pltpu.async_copy(ep.at[eidb], epb, sem2)
                    c1.wait()
                    c2.wait()

                    def edge_body(i, _):
                        dl = _ld_scalar(dlb, i)
                        for k in range(8):
                            sl = pl.ds(k * 16, 16)
                            v = xrb[i, sl] + epb[i, sl]
                            plsc.addupdate(s1s.at[dl, sl], v)
                            plsc.addupdate(s2s.at[dl, sl], v * v)
                            mns[dl, sl] = jnp.minimum(mns[dl, sl], v)
                            mxs[dl, sl] = jnp.maximum(mxs[dl, sl], v)
                        dbase = (dl // 16) * 16
                        dsl = pl.ds(dbase, 16)
                        dold = degs[dsl]
                        degs[dsl] = dold + jnp.where(
                            lax.iota(_i32, 16) == dl - dbase, 1.0, 0.0)
                        return 0

                    lax.fori_loop(0, m, edge_body, 0)
                    return 0

                lax.fori_loop(0, nfl, flush_body, 0)
                return 0

            lax.fori_loop(0, NW, src_body, 0)

            nbase = b * R
            pltpu.sync_copy(degs, deg.at[pl.ds(nbase, R)])
            pltpu.sync_copy(s1s, s1.at[pl.ds(nbase, R), :])
            pltpu.sync_copy(s2s, s2.at[pl.ds(nbase, R), :])
            pltpu.sync_copy(mns, mn.at[pl.ds(nbase, R), :])
            pltpu.sync_copy(mxs, mx.at[pl.ds(nbase, R), :])


def _segment_acc(xr, ep, route, counts):
    mesh = plsc.VectorSubcoreMesh(core_axis_name="c", subcore_axis_name="s", num_cores=NC, num_subcores=NS)
    f = pl.kernel(
        _acc_body,
        out_type=[jax.ShapeDtypeStruct((NPAD,), _f32),
                  jax.ShapeDtypeStruct((NPAD, 128), _f32),
                  jax.ShapeDtypeStruct((NPAD, 128), _f32),
                  jax.ShapeDtypeStruct((NPAD, 128), _f32),
                  jax.ShapeDtypeStruct((NPAD, 128), _f32)],
        mesh=mesh,
        scratch_types=[
            pltpu.VMEM((NW * 64 + 16,), _i32),
            pltpu.VMEM((2, FL), _i32),
            pltpu.VMEM((FL,), _i32),
            pltpu.VMEM((FL,), _i32),
            pltpu.VMEM((FL + 16,), _i32),
            pltpu.VMEM((FL, 128), _f32),
            pltpu.VMEM((FL, 128), _f32),
            pltpu.VMEM((R, 128), _f32),
            pltpu.VMEM((R, 128), _f32),
            pltpu.VMEM((R, 128), _f32),
            pltpu.VMEM((R, 128), _f32),
            pltpu.VMEM((R,), _f32),
            pltpu.SemaphoreType.DMA,
            pltpu.SemaphoreType.DMA,
        ],
    )
    return f(xr, ep, route, counts.reshape(NW * 64))


_ROUTE_SHAPE = (NW, NB, NFLUSH, 2, FL)


_BN = 1000


def _nodes_body(act, hpre, scale, shift, wct, wrt, h_out, xc_out, xr_out):
    h = hpre[...]
    if act:
        h = jnp.maximum(h * scale[...] + shift[...], 0.0)
    h_out[...] = h
    xc_out[...] = jnp.dot(h, wct[...], preferred_element_type=_f32)
    xr_out[...] = jnp.dot(h, wrt[...], preferred_element_type=_f32)


def _nodes(hpre, scale, shift, wct, wrt, act):
    fullspec = pl.BlockSpec((1, 128), lambda i: (0, 0))
    wspec = pl.BlockSpec((128, 128), lambda i: (0, 0))
    rspec = pl.BlockSpec((_BN, 128), lambda i: (i, 0))
    return pl.pallas_call(
        functools.partial(_nodes_body, act),
        grid=(N // _BN,),
        in_specs=[rspec, fullspec, fullspec, wspec, wspec],
        out_specs=[rspec, rspec, rspec],
        out_shape=[jax.ShapeDtypeStruct((N, 128), _f32)] * 3,
    )(hpre, scale, shift, wct, wrt)


def _ep_body(ea, wet, bet, wept, bpret, o1, o2, o3, o4):
    outs = [o1, o2, o3, o4]
    for l in range(4):
        e = (jnp.dot(ea[...], wet[l], preferred_element_type=_f32)
             + bet[l])
        outs[l][...] = (jnp.dot(e, wept[l],
                                preferred_element_type=_f32) + bpret[l])


def _edge_mlp(ea, wet, bet, wept, bpret):
    BE = 4000
    return pl.pallas_call(
        _ep_body,
        grid=(E // BE,),
        in_specs=[pl.BlockSpec((BE, 4), lambda i: (i, 0)),
                  pl.BlockSpec((4, 4, 128), lambda i: (0, 0, 0)),
                  pl.BlockSpec((4, 1, 128), lambda i: (0, 0, 0)),
                  pl.BlockSpec((4, 128, 128), lambda i: (0, 0, 0)),
                  pl.BlockSpec((4, 1, 128), lambda i: (0, 0, 0))],
        out_specs=[pl.BlockSpec((BE, 128), lambda i: (i, 0))] * 4,
        out_shape=[jax.ShapeDtypeStruct((E, 128), _f32)] * 4,
    )(ea, wet, bet, wept, bpret)


def _combine_body(h, xc, deg, s1, s2, mn, mx, at_, b1t, b2t, b3t, wlt,
                  bpost, blin, out, bn_s, bn_q):
    d = jnp.maximum(deg[...], 1.0)
    dinv = 1.0 / d
    pos = deg[...] > 0.0
    s1n = s1[...] * dinv
    mean = jnp.where(pos, xc[...] + s1n, 0.0)
    var = jnp.maximum(s2[...] * dinv - s1n * s1n, 0.0)
    std = jnp.sqrt(var + 1e-5)
    mnv = jnp.where(pos, xc[...] + mn[...], 0.0)
    mxv = jnp.where(pos, xc[...] + mx[...], 0.0)
    agg = jnp.concatenate([mean, mnv, mxv, std], axis=-1)
    logd = jnp.log(d + 1.0)
    amp = logd * (1.0 / AVG_LOG)
    att = AVG_LOG / logd
    t = (jnp.dot(h[...], at_[...], preferred_element_type=_f32)
         + jnp.dot(agg, b1t[...], preferred_element_type=_f32)
         + jnp.dot(agg * amp, b2t[...], preferred_element_type=_f32)
         + jnp.dot(agg * att, b3t[...], preferred_element_type=_f32)
         + bpost[...])
    o = jnp.dot(t, wlt[...], preferred_element_type=_f32) + blin[...]
    out[...] = o

    @pl.when(pl.program_id(0) == 0)
    def _():
        bn_s[...] = jnp.zeros_like(bn_s)
        bn_q[...] = jnp.zeros_like(bn_q)

    bn_s[...] += jnp.sum(o, axis=0, keepdims=True)
    bn_q[...] += jnp.sum(o * o, axis=0, keepdims=True)


def _combine(h, xc, deg, s1, s2, mn, mx, at_, b1t, b2t, b3t, wlt, bpost,
             blin):
    rspec = pl.BlockSpec((_BN, 128), lambda i: (i, 0))
    return pl.pallas_call(
        _combine_body,
        grid=(N // _BN,),
        in_specs=[rspec, rspec,
                  pl.BlockSpec((_BN, 1), lambda i: (i, 0)),
                  rspec, rspec, rspec, rspec,
                  pl.BlockSpec((128, 128), lambda i: (0, 0)),
                  pl.BlockSpec((512, 128), lambda i: (0, 0)),
                  pl.BlockSpec((512, 128), lambda i: (0, 0)),
                  pl.BlockSpec((512, 128), lambda i: (0, 0)),
                  pl.BlockSpec((128, 128), lambda i: (0, 0)),
                  pl.BlockSpec((1, 128), lambda i: (0, 0)),
                  pl.BlockSpec((1, 128), lambda i: (0, 0))],
        out_specs=[rspec,
                   pl.BlockSpec((1, 128), lambda i: (0, 0)),
                   pl.BlockSpec((1, 128), lambda i: (0, 0))],
        out_shape=[jax.ShapeDtypeStruct((N, 128), _f32),
                   jax.ShapeDtypeStruct((1, 128), _f32),
                   jax.ShapeDtypeStruct((1, 128), _f32)],
    )(h, xc, deg, s1, s2, mn, mx, at_, b1t, b2t, b3t, wlt, bpost, blin)


def _bnparams_body(bs, bq, g, bta, scale, shift):
    mu = bs[...] * (1.0 / N)
    var = bq[...] * (1.0 / N) - mu * mu
    sc = g[...] * lax.rsqrt(var + 1e-5)
    scale[...] = sc
    shift[...] = bta[...] - mu * sc


def _bnparams(bs, bq, g, bta):
    spec = pl.BlockSpec((1, 128), lambda: (0, 0))
    return pl.pallas_call(
        _bnparams_body,
        in_specs=[spec] * 4,
        out_specs=[spec] * 2,
        out_shape=[jax.ShapeDtypeStruct((1, 128), _f32)] * 2,
    )(bs, bq, g, bta)


def _pool_body(hpre, scale, shift, bid, mxo, smo, cnto):
    h = jnp.maximum(hpre[...] * scale[...] + shift[...], 0.0)
    b = bid[...]

    @pl.when(pl.program_id(0) == 0)
    def _():
        mxo[...] = jnp.full_like(mxo, -jnp.inf)
        smo[...] = jnp.zeros_like(smo)
        cnto[...] = jnp.zeros_like(cnto)

    lo = b[0, 0]
    hi = b[_BN - 1, 0]
    for g in range(G):
        @pl.when(jnp.logical_and(lo <= g, g <= hi))
        def _():
            msk = b == g
            hm = h + jnp.where(msk, 0.0, -jnp.inf)
            mxo[g, :] = jnp.maximum(mxo[g, :], jnp.max(hm, axis=0))
            hs = jnp.where(msk, h, 0.0)
            smo[g, :] += jnp.sum(hs, axis=0)
            cnto[g, :] += jnp.full((128,), 1.0) * jnp.sum(msk.astype(_f32))


def _pool(hpre, scale, shift, bid2d):
    return pl.pallas_call(
        _pool_body,
        grid=(N // _BN,),
        in_specs=[pl.BlockSpec((_BN, 128), lambda i: (i, 0)),
                  pl.BlockSpec((1, 128), lambda i: (0, 0)),
                  pl.BlockSpec((1, 128), lambda i: (0, 0)),
                  pl.BlockSpec((_BN, 1), lambda i: (i, 0))],
        out_specs=[pl.BlockSpec((G, 128), lambda i: (0, 0)),
                   pl.BlockSpec((G, 128), lambda i: (0, 0)),
                   pl.BlockSpec((G, 128), lambda i: (0, 0))],
        out_shape=[jax.ShapeDtypeStruct((G, 128), _f32),
                   jax.ShapeDtypeStruct((G, 128), _f32),
                   jax.ShapeDtypeStruct((G, 128), _f32)],
    )(hpre, scale, shift, bid2d)


def _head_body(mx, sm, cnt, w1t, b1, w2t, b2, w3t, b3, out):
    mxf = jnp.where(jnp.isfinite(mx[...]), mx[...], 0.0)
    mean = sm[...] / jnp.maximum(cnt[...], 1.0)
    z = jnp.concatenate([mxf, mean], axis=1)
    z = jnp.maximum(jnp.dot(z, w1t[...], preferred_element_type=_f32)
                    + b1[...], 0.0)
    z = jnp.maximum(jnp.dot(z, w2t[...], preferred_element_type=_f32)
                    + b2[...], 0.0)
    out[...] = jnp.dot(z, w3t[...], preferred_element_type=_f32) + b3[...]


def _head(mx, sm, cnt, w1t, b1, w2t, b2, w3t, b3):
    return pl.pallas_call(
        _head_body,
        in_specs=[pl.BlockSpec((G, 128), lambda: (0, 0)),
                  pl.BlockSpec((G, 128), lambda: (0, 0)),
                  pl.BlockSpec((G, 128), lambda: (0, 0)),
                  pl.BlockSpec((256, 128), lambda: (0, 0)),
                  pl.BlockSpec((1, 128), lambda: (0, 0)),
                  pl.BlockSpec((128, 64), lambda: (0, 0)),
                  pl.BlockSpec((1, 64), lambda: (0, 0)),
                  pl.BlockSpec((64, 128), lambda: (0, 0)),
                  pl.BlockSpec((1, 128), lambda: (0, 0))],
        out_specs=pl.BlockSpec((G, 128), lambda: (0, 0)),
        out_shape=jax.ShapeDtypeStruct((G, 128), _f32),
    )(mx, sm, cnt, w1t, b1, w2t, b2, w3t, b3)


def kernel(x, edge_index, edge_attr, intarna_energy, batch, covalent_edges,
           dropout_conv_1_2, dropout_conv_rest, params):
    del intarna_energy, dropout_conv_1_2, dropout_conv_rest

    lw = []
    wet, bet, wept, bpret = [], [], [], []
    for i in range(1, 5):
        p = params["conv%d" % i]
        wpre = p["Wpre"]
        wc, wr, wep = wpre[:, 0:128], wpre[:, 128:256], wpre[:, 256:384]
        wet.append(p["We"].T)
        bet.append(p["be"].reshape(1, 128))
        wept.append(wep.T)
        bpret.append(p["bpre"].reshape(1, 128))
        wpost = p["Wpost"]
        lw.append(dict(
            wct=wc.T, wrt=wr.T,
            at_=wpost[:, 0:128].T,
            b1t=wpost[:, 128:640].T,
            b2t=wpost[:, 640:1152].T,
            b3t=wpost[:, 1152:1664].T,
            wlt=p["Wlin"].T,
            bpost=p["bpost"].reshape(1, 128),
            blin=p["blin"].reshape(1, 128),
            bn_g=p["bn_g"].reshape(1, 128),
            bn_b=p["bn_b"].reshape(1, 128),
        ))
    wet = jnp.stack(wet)
    bet = jnp.stack(bet)
    wept = jnp.stack(wept)
    bpret = jnp.stack(bpret)

    ei0 = edge_index[0]
    ei1 = edge_index[1]
    cov32 = covalent_edges.astype(_i32)

    route, counts = _route_edges(ei0, ei1, cov32)

    eps = _edge_mlp(edge_attr, wet, bet, wept, bpret)

    ones = jnp.ones((1, 128), _f32)
    zeros = jnp.zeros((1, 128), _f32)

    hpre = x
    scale, shift = ones, zeros
    for l in range(4):
        w = lw[l]
        h, xc, xr = _nodes(hpre, scale, shift, w["wct"], w["wrt"], l > 0)
        deg, s1, s2, mn, mx = _segment_acc(xr, eps[l], route, counts)
        deg2 = deg[:N].reshape(N, 1)
        hpre, bs, bq = _combine(h, xc, deg2, s1[:N], s2[:N], mn[:N], mx[:N],
                                w["at_"], w["b1t"], w["b2t"], w["b3t"],
                                w["wlt"], w["bpost"], w["blin"])
        scale, shift = _bnparams(bs, bq, w["bn_g"], w["bn_b"])

    bid2d = batch.astype(_i32).reshape(N, 1)
    mxp, smp, cntp = _pool(hpre, scale, shift, bid2d)

    w1t = params["lin1_W"].T
    w2t = params["lin2_W"].T
    w3t = jnp.concatenate(
        [params["lin3_W"], jnp.zeros((126, 64), _f32)], axis=0).T
    b2p = params["lin2_b"].reshape(1, 64)
    b3p = jnp.concatenate(
        [params["lin3_b"], jnp.zeros((126,), _f32)]).reshape(1, 128)
    out = _head(mxp, smp, cntp, w1t, params["lin1_b"].reshape(1, 128),
                w2t, b2p, w3t, b3p)
    return out[:, :2]
